# baseline (speedup 1.0000x reference)
import jax
import jax.numpy as jnp
from jax import lax
from jax.experimental import pallas as pl
from jax.experimental.pallas import tpu as pltpu
from jax.experimental.pallas import tpu_sc as plsc

N_FIELDS_S = 26
VOCAB_S = 100000
EMB_S = 32
N_CONT_S = 13
CHUNK = 32
LANES = 16
N_WORKERS = 32
CONT_OFF = N_FIELDS_S * EMB_S
N_GATHERS = N_FIELDS_S * CHUNK // 64
PAIR_ROWS = N_FIELDS_S // 2


def _tc_relayout(tables_t):
    F, E, V = tables_t.shape
    W = 512
    grid = (F, pl.cdiv(V, W))

    def body(in_ref, out_ref):
        out_ref[0] = in_ref[0].T

    return pl.pallas_call(
        body,
        grid=grid,
        in_specs=[pl.BlockSpec((1, E, W), lambda c, j: (c, 0, j))],
        out_specs=pl.BlockSpec((1, W, E), lambda c, j: (c, j, 0)),
        out_shape=jax.ShapeDtypeStruct((F, V, E), jnp.float32),
    )(tables_t)


def _lane_broadcast(v, lane):
    idx = jnp.full((LANES, 1), lane, jnp.int32)
    dnums = lax.GatherDimensionNumbers(
        offset_dims=(), collapsed_slice_dims=(0,), start_index_map=(0,))
    return lax.gather(v, idx, dnums, slice_sizes=(1,),
                      mode=lax.GatherScatterMode.PROMISE_IN_BOUNDS)


def _body(tabf_hbm, xs_fm_hbm, offs_hbm, xcp_hbm, fe_hbm, out_hbm,
          idx0, idx1, rows0, rows1, cont0, cont1, xcp0, xcp1, offs_v, fe_v,
          gsem0, gsem1, osem0, osem1):
    B = out_hbm.shape[0]
    rows_per_w = B // N_WORKERS
    n_chunks = rows_per_w // CHUNK

    wid = lax.axis_index("s") * 2 + lax.axis_index("c")
    chunk_base = wid * n_chunks

    pltpu.sync_copy(fe_hbm, fe_v)
    pltpu.sync_copy(offs_hbm, offs_v)

    idx_b = (idx0, idx1)
    rows_b = (rows0, rows1)
    cont_b = (cont0, cont1)
    xcp_b = (xcp0, xcp1)
    gsem_b = (gsem0, gsem1)
    osem_b = (osem0, osem1)

    def drain_out(p):
        rows_v, cont_v, osem = rows_b[p], cont_b[p], osem_b[p]
        for f in range(N_FIELDS_S):
            pltpu.make_async_copy(
                rows_v.at[pl.ds(f * CHUNK, CHUNK)],
                out_hbm.at[pl.ds(0, CHUNK), pl.ds(f * EMB_S, EMB_S)],
                osem).wait()
        pltpu.make_async_copy(
            cont_v,
            out_hbm.at[pl.ds(0, CHUNK), pl.ds(CONT_OFF, N_CONT_S * EMB_S)],
            osem).wait()

    def run_chunk(g, p, wait_out):
        idx_v, rows_v, cont_v = idx_b[p], rows_b[p], cont_b[p]
        xcp_v, gsem, osem = xcp_b[p], gsem_b[p], osem_b[p]
        ck = chunk_base + g
        c0 = ck * CHUNK

        if wait_out:
            drain_out(p)

        pltpu.sync_copy(xs_fm_hbm.at[ck], idx_v)
        pltpu.sync_copy(xcp_hbm.at[pl.ds(c0, CHUNK)], xcp_v)

        for j in range(PAIR_ROWS):
            for k in range(4):
                sl = pl.ds(k * LANES, LANES)
                idx_v[j, sl] = idx_v[j, sl] + offs_v[j, sl]

        cps = [
            pltpu.async_copy(
                tabf_hbm.at[idx_v.at[j]],
                rows_v.at[pl.ds(j * 64, 64)],
                gsem)
            for j in range(N_GATHERS)
        ]

        def b_body(b, c):
            v = xcp_v[b, :]
            for cv in range(N_CONT_S):
                bc = _lane_broadcast(v, cv)
                o = cv * EMB_S
                cont_v[b, pl.ds(o, LANES)] = bc * fe_v[cv, pl.ds(0, LANES)]
                cont_v[b, pl.ds(o + LANES, LANES)] = (
                    bc * fe_v[cv, pl.ds(LANES, LANES)])
            return c

        lax.fori_loop(0, CHUNK, b_body, 0)

        for cp in cps:
            cp.wait()

        for f in range(N_FIELDS_S):
            pltpu.async_copy(
                rows_v.at[pl.ds(f * CHUNK, CHUNK)],
                out_hbm.at[pl.ds(c0, CHUNK), pl.ds(f * EMB_S, EMB_S)],
                osem)
        pltpu.async_copy(
            cont_v,
            out_hbm.at[pl.ds(c0, CHUNK), pl.ds(CONT_OFF, N_CONT_S * EMB_S)],
            osem)

    run_chunk(0, 0, False)
    run_chunk(1, 1, False)

    def outer(i, carry):
        run_chunk(2 * i, 0, True)
        run_chunk(2 * i + 1, 1, True)
        return carry

    lax.fori_loop(1, n_chunks // 2, outer, 0)

    drain_out(0)
    drain_out(1)


def kernel(x_sparse, x_cont, tables, field_embeddings):
    B, F = x_sparse.shape
    V, E = tables.shape[1], tables.shape[2]
    C = x_cont.shape[1]
    n_chunks_total = B // CHUNK

    tabf = _tc_relayout(tables.transpose(0, 2, 1)).reshape(F * V, E)
    xs_fm = (x_sparse.T.reshape(F, n_chunks_total, CHUNK)
             .transpose(1, 0, 2).reshape(n_chunks_total, F // 2, 64))
    offs = jnp.repeat(jnp.arange(F, dtype=jnp.int32) * V, CHUNK)
    offs = offs.reshape(F // 2, 64)
    xcp = jnp.pad(x_cont, ((0, 0), (0, LANES - C)))

    mesh = plsc.VectorSubcoreMesh(core_axis_name="c", subcore_axis_name="s")
    run = pl.kernel(
        _body,
        out_type=jax.ShapeDtypeStruct((B, (F + C) * E), jnp.float32),
        mesh=mesh,
        scratch_types=[
            pltpu.VMEM((F // 2, 64), jnp.int32),
            pltpu.VMEM((F // 2, 64), jnp.int32),
            pltpu.VMEM((F * CHUNK, E), jnp.float32),
            pltpu.VMEM((F * CHUNK, E), jnp.float32),
            pltpu.VMEM((CHUNK, C * E), jnp.float32),
            pltpu.VMEM((CHUNK, C * E), jnp.float32),
            pltpu.VMEM((CHUNK, LANES), jnp.float32),
            pltpu.VMEM((CHUNK, LANES), jnp.float32),
            pltpu.VMEM((F // 2, 64), jnp.int32),
            pltpu.VMEM((C, E), jnp.float32),
            pltpu.SemaphoreType.DMA,
            pltpu.SemaphoreType.DMA,
            pltpu.SemaphoreType.DMA,
            pltpu.SemaphoreType.DMA,
        ],
        compiler_params=pltpu.CompilerParams(use_tc_tiling_on_sc=False),
    )
    return run(tabf, xs_fm, offs, xcp, field_embeddings)


# speedup vs baseline: 2.1332x; 2.1332x over previous
import jax
import jax.numpy as jnp
from jax import lax
from jax.experimental import pallas as pl
from jax.experimental.pallas import tpu as pltpu
from jax.experimental.pallas import tpu_sc as plsc

N_FIELDS_S = 26
VOCAB_S = 100000
EMB_S = 32
N_CONT_S = 13
CHUNK = 32
LANES = 16
N_WORKERS = 32
CONT_OFF = N_FIELDS_S * EMB_S
N_GATHERS = N_FIELDS_S * CHUNK // 64
PAIR_ROWS = N_FIELDS_S // 2


def _tc_relayout(tables_t):
    F, E, V = tables_t.shape
    W = 4096
    grid = (F, pl.cdiv(V, W))

    def body(in_ref, out_ref):
        out_ref[0] = in_ref[0].T

    return pl.pallas_call(
        body,
        grid=grid,
        in_specs=[pl.BlockSpec((1, E, W), lambda c, j: (c, 0, j))],
        out_specs=pl.BlockSpec((1, W, E), lambda c, j: (c, j, 0)),
        out_shape=jax.ShapeDtypeStruct((F, V, E), jnp.float32),
    )(tables_t)


def _lane_broadcast(v, lane):
    idx = jnp.full((LANES, 1), lane, jnp.int32)
    dnums = lax.GatherDimensionNumbers(
        offset_dims=(), collapsed_slice_dims=(0,), start_index_map=(0,))
    return lax.gather(v, idx, dnums, slice_sizes=(1,),
                      mode=lax.GatherScatterMode.PROMISE_IN_BOUNDS)


def _body(tabf_hbm, xs_fm_hbm, offs_hbm, xcp_hbm, fe_hbm, out_hbm,
          idx0, idx1, rows0, rows1, cont0, cont1, xcp0, xcp1, offs_v, fe_v,
          gsem0, gsem1, osem0, osem1):
    B = out_hbm.shape[0]
    rows_per_w = B // N_WORKERS
    n_chunks = rows_per_w // CHUNK

    wid = lax.axis_index("s") * 2 + lax.axis_index("c")
    chunk_base = wid * n_chunks

    pltpu.sync_copy(fe_hbm, fe_v)
    pltpu.sync_copy(offs_hbm, offs_v)

    idx_b = (idx0, idx1)
    rows_b = (rows0, rows1)
    cont_b = (cont0, cont1)
    xcp_b = (xcp0, xcp1)
    gsem_b = (gsem0, gsem1)
    osem_b = (osem0, osem1)

    def drain_out(p):
        rows_v, cont_v, osem = rows_b[p], cont_b[p], osem_b[p]
        for f in range(N_FIELDS_S):
            pltpu.make_async_copy(
                rows_v.at[pl.ds(f * CHUNK, CHUNK)],
                out_hbm.at[pl.ds(0, CHUNK), pl.ds(f * EMB_S, EMB_S)],
                osem).wait()
        pltpu.make_async_copy(
            cont_v,
            out_hbm.at[pl.ds(0, CHUNK), pl.ds(CONT_OFF, N_CONT_S * EMB_S)],
            osem).wait()

    def run_chunk(g, p, wait_out):
        idx_v, rows_v, cont_v = idx_b[p], rows_b[p], cont_b[p]
        xcp_v, gsem, osem = xcp_b[p], gsem_b[p], osem_b[p]
        ck = chunk_base + g
        c0 = ck * CHUNK

        if wait_out:
            drain_out(p)

        pltpu.sync_copy(xs_fm_hbm.at[ck], idx_v)
        pltpu.sync_copy(xcp_hbm.at[pl.ds(c0, CHUNK)], xcp_v)

        for j in range(PAIR_ROWS):
            for k in range(4):
                sl = pl.ds(k * LANES, LANES)
                idx_v[j, sl] = idx_v[j, sl] + offs_v[j, sl]

        cps = [
            pltpu.async_copy(
                tabf_hbm.at[idx_v.at[j]],
                rows_v.at[pl.ds(j * 64, 64)],
                gsem)
            for j in range(N_GATHERS)
        ]

        def b_body(b, c):
            v = xcp_v[b, :]
            for cv in range(N_CONT_S):
                bc = _lane_broadcast(v, cv)
                o = cv * EMB_S
                cont_v[b, pl.ds(o, LANES)] = bc * fe_v[cv, pl.ds(0, LANES)]
                cont_v[b, pl.ds(o + LANES, LANES)] = (
                    bc * fe_v[cv, pl.ds(LANES, LANES)])
            return c

        lax.fori_loop(0, CHUNK, b_body, 0)

        for cp in cps:
            cp.wait()

        for f in range(N_FIELDS_S):
            pltpu.async_copy(
                rows_v.at[pl.ds(f * CHUNK, CHUNK)],
                out_hbm.at[pl.ds(c0, CHUNK), pl.ds(f * EMB_S, EMB_S)],
                osem)
        pltpu.async_copy(
            cont_v,
            out_hbm.at[pl.ds(c0, CHUNK), pl.ds(CONT_OFF, N_CONT_S * EMB_S)],
            osem)

    run_chunk(0, 0, False)
    run_chunk(1, 1, False)

    def outer(i, carry):
        run_chunk(2 * i, 0, True)
        run_chunk(2 * i + 1, 1, True)
        return carry

    lax.fori_loop(1, n_chunks // 2, outer, 0)

    drain_out(0)
    drain_out(1)


def kernel(x_sparse, x_cont, tables, field_embeddings):
    B, F = x_sparse.shape
    V, E = tables.shape[1], tables.shape[2]
    C = x_cont.shape[1]
    n_chunks_total = B // CHUNK

    tabf = _tc_relayout(tables.transpose(0, 2, 1)).reshape(F * V, E)
    xs_fm = (x_sparse.T.reshape(F, n_chunks_total, CHUNK)
             .transpose(1, 0, 2).reshape(n_chunks_total, F // 2, 64))
    offs = jnp.repeat(jnp.arange(F, dtype=jnp.int32) * V, CHUNK)
    offs = offs.reshape(F // 2, 64)
    xcp = jnp.pad(x_cont, ((0, 0), (0, LANES - C)))

    mesh = plsc.VectorSubcoreMesh(core_axis_name="c", subcore_axis_name="s")
    run = pl.kernel(
        _body,
        out_type=jax.ShapeDtypeStruct((B, (F + C) * E), jnp.float32),
        mesh=mesh,
        scratch_types=[
            pltpu.VMEM((F // 2, 64), jnp.int32),
            pltpu.VMEM((F // 2, 64), jnp.int32),
            pltpu.VMEM((F * CHUNK, E), jnp.float32),
            pltpu.VMEM((F * CHUNK, E), jnp.float32),
            pltpu.VMEM((CHUNK, C * E), jnp.float32),
            pltpu.VMEM((CHUNK, C * E), jnp.float32),
            pltpu.VMEM((CHUNK, LANES), jnp.float32),
            pltpu.VMEM((CHUNK, LANES), jnp.float32),
            pltpu.VMEM((F // 2, 64), jnp.int32),
            pltpu.VMEM((C, E), jnp.float32),
            pltpu.SemaphoreType.DMA,
            pltpu.SemaphoreType.DMA,
            pltpu.SemaphoreType.DMA,
            pltpu.SemaphoreType.DMA,
        ],
        compiler_params=pltpu.CompilerParams(use_tc_tiling_on_sc=False),
    )
    return run(tabf, xs_fm, offs, xcp, field_embeddings)


# speedup vs baseline: 6.2384x; 2.9245x over previous
import jax
import jax.numpy as jnp
from jax import lax
from jax.experimental import pallas as pl
from jax.experimental.pallas import tpu as pltpu
from jax.experimental.pallas import tpu_sc as plsc

N_FIELDS_S = 26
VOCAB_S = 100000
EMB_S = 32
N_CONT_S = 13
CHUNK = 32
LANES = 16
N_WORKERS = 32
CONT_OFF = N_FIELDS_S * EMB_S
N_GATHERS = N_FIELDS_S * CHUNK // 64
PAIR_ROWS = N_FIELDS_S // 2


def _tc_relayout(tables_t):
    F, E, V = tables_t.shape
    GF = 4
    G = -(-F // GF)
    W = 2048
    grid = (G, pl.cdiv(V, W))

    def body(in_ref, out_ref):
        x = jnp.concatenate([in_ref[a] for a in range(GF)], axis=0)
        out_ref[0] = x.T

    return pl.pallas_call(
        body,
        grid=grid,
        in_specs=[pl.BlockSpec((GF, E, W), lambda g, j: (g, 0, j))],
        out_specs=pl.BlockSpec((1, W, GF * E), lambda g, j: (g, j, 0)),
        out_shape=jax.ShapeDtypeStruct((G, V, GF * E), jnp.float32),
    )(tables_t)


def _lane_broadcast(v, lane):
    idx = jnp.full((LANES, 1), lane, jnp.int32)
    dnums = lax.GatherDimensionNumbers(
        offset_dims=(), collapsed_slice_dims=(0,), start_index_map=(0,))
    return lax.gather(v, idx, dnums, slice_sizes=(1,),
                      mode=lax.GatherScatterMode.PROMISE_IN_BOUNDS)


def _body(tabf_hbm, xs_fm_hbm, offs_hbm, xcp_hbm, fe_hbm, out_hbm,
          idx0, idx1, rows0, rows1, cont0, cont1, xcp0, xcp1, offs_v, fe_v,
          gsem0, gsem1, osem0, osem1):
    B = out_hbm.shape[0]
    rows_per_w = B // N_WORKERS
    n_chunks = rows_per_w // CHUNK

    wid = lax.axis_index("s") * 2 + lax.axis_index("c")
    chunk_base = wid * n_chunks

    pltpu.sync_copy(fe_hbm, fe_v)
    pltpu.sync_copy(offs_hbm, offs_v)

    idx_b = (idx0, idx1)
    rows_b = (rows0, rows1)
    cont_b = (cont0, cont1)
    xcp_b = (xcp0, xcp1)
    gsem_b = (gsem0, gsem1)
    osem_b = (osem0, osem1)

    def drain_out(p):
        rows_v, cont_v, osem = rows_b[p], cont_b[p], osem_b[p]
        for f in range(N_FIELDS_S):
            pltpu.make_async_copy(
                rows_v.at[pl.ds(f * CHUNK, CHUNK)],
                out_hbm.at[pl.ds(0, CHUNK), pl.ds(f * EMB_S, EMB_S)],
                osem).wait()
        pltpu.make_async_copy(
            cont_v,
            out_hbm.at[pl.ds(0, CHUNK), pl.ds(CONT_OFF, N_CONT_S * EMB_S)],
            osem).wait()

    def run_chunk(g, p, wait_out):
        idx_v, rows_v, cont_v = idx_b[p], rows_b[p], cont_b[p]
        xcp_v, gsem, osem = xcp_b[p], gsem_b[p], osem_b[p]
        ck = chunk_base + g
        c0 = ck * CHUNK

        if wait_out:
            drain_out(p)

        pltpu.sync_copy(xs_fm_hbm.at[ck], idx_v)
        pltpu.sync_copy(xcp_hbm.at[pl.ds(c0, CHUNK)], xcp_v)

        for j in range(PAIR_ROWS):
            for k in range(4):
                sl = pl.ds(k * LANES, LANES)
                idx_v[j, sl] = idx_v[j, sl] * 4 + offs_v[j, sl]

        cps = [
            pltpu.async_copy(
                tabf_hbm.at[idx_v.at[j]],
                rows_v.at[pl.ds(j * 64, 64)],
                gsem)
            for j in range(N_GATHERS)
        ]

        def b_body(b, c):
            v = xcp_v[b, :]
            for cv in range(N_CONT_S):
                bc = _lane_broadcast(v, cv)
                o = cv * EMB_S
                cont_v[b, pl.ds(o, LANES)] = bc * fe_v[cv, pl.ds(0, LANES)]
                cont_v[b, pl.ds(o + LANES, LANES)] = (
                    bc * fe_v[cv, pl.ds(LANES, LANES)])
            return c

        lax.fori_loop(0, CHUNK, b_body, 0)

        for cp in cps:
            cp.wait()

        for f in range(N_FIELDS_S):
            pltpu.async_copy(
                rows_v.at[pl.ds(f * CHUNK, CHUNK)],
                out_hbm.at[pl.ds(c0, CHUNK), pl.ds(f * EMB_S, EMB_S)],
                osem)
        pltpu.async_copy(
            cont_v,
            out_hbm.at[pl.ds(c0, CHUNK), pl.ds(CONT_OFF, N_CONT_S * EMB_S)],
            osem)

    run_chunk(0, 0, False)
    run_chunk(1, 1, False)

    def outer(i, carry):
        run_chunk(2 * i, 0, True)
        run_chunk(2 * i + 1, 1, True)
        return carry

    lax.fori_loop(1, n_chunks // 2, outer, 0)

    drain_out(0)
    drain_out(1)


def kernel(x_sparse, x_cont, tables, field_embeddings):
    B, F = x_sparse.shape
    V, E = tables.shape[1], tables.shape[2]
    C = x_cont.shape[1]
    n_chunks_total = B // CHUNK

    tab3 = _tc_relayout(tables.transpose(0, 2, 1))
    tabf = tab3.reshape(tab3.shape[0] * V * 4, E)
    xs_fm = (x_sparse.T.reshape(F, n_chunks_total, CHUNK)
             .transpose(1, 0, 2).reshape(n_chunks_total, F // 2, 64))
    fld = jnp.arange(F, dtype=jnp.int32)
    bases = (fld // 4) * (4 * V) + (fld % 4)
    offs = jnp.repeat(bases, CHUNK).reshape(F // 2, 64)
    xcp = jnp.pad(x_cont, ((0, 0), (0, LANES - C)))

    mesh = plsc.VectorSubcoreMesh(core_axis_name="c", subcore_axis_name="s")
    run = pl.kernel(
        _body,
        out_type=jax.ShapeDtypeStruct((B, (F + C) * E), jnp.float32),
        mesh=mesh,
        scratch_types=[
            pltpu.VMEM((F // 2, 64), jnp.int32),
            pltpu.VMEM((F // 2, 64), jnp.int32),
            pltpu.VMEM((F * CHUNK, E), jnp.float32),
            pltpu.VMEM((F * CHUNK, E), jnp.float32),
            pltpu.VMEM((CHUNK, C * E), jnp.float32),
            pltpu.VMEM((CHUNK, C * E), jnp.float32),
            pltpu.VMEM((CHUNK, LANES), jnp.float32),
            pltpu.VMEM((CHUNK, LANES), jnp.float32),
            pltpu.VMEM((F // 2, 64), jnp.int32),
            pltpu.VMEM((C, E), jnp.float32),
            pltpu.SemaphoreType.DMA,
            pltpu.SemaphoreType.DMA,
            pltpu.SemaphoreType.DMA,
            pltpu.SemaphoreType.DMA,
        ],
        compiler_params=pltpu.CompilerParams(use_tc_tiling_on_sc=False),
    )
    return run(tabf, xs_fm, offs, xcp, field_embeddings)


# speedup vs baseline: 7.7716x; 1.2458x over previous
import jax
import jax.numpy as jnp
from jax import lax
from jax.experimental import pallas as pl
from jax.experimental.pallas import tpu as pltpu
from jax.experimental.pallas import tpu_sc as plsc

N_FIELDS_S = 26
VOCAB_S = 100000
EMB_S = 32
N_CONT_S = 13
CHUNK = 32
LANES = 16
N_WORKERS = 32
CONT_OFF = N_FIELDS_S * EMB_S
N_GATHERS = N_FIELDS_S * CHUNK // 64
PAIR_ROWS = N_FIELDS_S // 2


def _tc_relayout(tables_t):
    F, E, V = tables_t.shape
    GF = 4
    G = -(-F // GF)
    W = 8192
    grid = (G, pl.cdiv(V, W))

    def body(in_ref, out_ref):
        x = jnp.concatenate([in_ref[a] for a in range(GF)], axis=0)
        out_ref[0] = x.T

    return pl.pallas_call(
        body,
        grid=grid,
        in_specs=[pl.BlockSpec((GF, E, W), lambda g, j: (g, 0, j))],
        out_specs=pl.BlockSpec((1, W, GF * E), lambda g, j: (g, j, 0)),
        out_shape=jax.ShapeDtypeStruct((G, V, GF * E), jnp.float32),
    )(tables_t)


def _lane_broadcast(v, lane):
    idx = jnp.full((LANES, 1), lane, jnp.int32)
    dnums = lax.GatherDimensionNumbers(
        offset_dims=(), collapsed_slice_dims=(0,), start_index_map=(0,))
    return lax.gather(v, idx, dnums, slice_sizes=(1,),
                      mode=lax.GatherScatterMode.PROMISE_IN_BOUNDS)


def _body(tabf_hbm, xs_fm_hbm, offs_hbm, xcp_hbm, fe_hbm, out_hbm,
          idx0, idx1, rows0, rows1, cont0, cont1, xcp0, xcp1, offs_v, fe_v,
          gsem0, gsem1, osem0, osem1):
    B = out_hbm.shape[0]
    rows_per_w = B // N_WORKERS
    n_chunks = rows_per_w // CHUNK

    wid = lax.axis_index("s") * 2 + lax.axis_index("c")
    chunk_base = wid * n_chunks

    pltpu.sync_copy(fe_hbm, fe_v)
    pltpu.sync_copy(offs_hbm, offs_v)

    idx_b = (idx0, idx1)
    rows_b = (rows0, rows1)
    cont_b = (cont0, cont1)
    xcp_b = (xcp0, xcp1)
    gsem_b = (gsem0, gsem1)
    osem_b = (osem0, osem1)

    def drain_out(p):
        rows_v, cont_v, osem = rows_b[p], cont_b[p], osem_b[p]
        for f in range(N_FIELDS_S):
            pltpu.make_async_copy(
                rows_v.at[pl.ds(f * CHUNK, CHUNK)],
                out_hbm.at[pl.ds(0, CHUNK), pl.ds(f * EMB_S, EMB_S)],
                osem).wait()
        pltpu.make_async_copy(
            cont_v,
            out_hbm.at[pl.ds(0, CHUNK), pl.ds(CONT_OFF, N_CONT_S * EMB_S)],
            osem).wait()

    def run_chunk(g, p, wait_out):
        idx_v, rows_v, cont_v = idx_b[p], rows_b[p], cont_b[p]
        xcp_v, gsem, osem = xcp_b[p], gsem_b[p], osem_b[p]
        ck = chunk_base + g
        c0 = ck * CHUNK

        if wait_out:
            drain_out(p)

        pltpu.sync_copy(xs_fm_hbm.at[ck], idx_v)
        pltpu.sync_copy(xcp_hbm.at[pl.ds(c0, CHUNK)], xcp_v)

        for j in range(PAIR_ROWS):
            for k in range(4):
                sl = pl.ds(k * LANES, LANES)
                idx_v[j, sl] = idx_v[j, sl] * 4 + offs_v[j, sl]

        cps = [
            pltpu.async_copy(
                tabf_hbm.at[idx_v.at[j]],
                rows_v.at[pl.ds(j * 64, 64)],
                gsem)
            for j in range(N_GATHERS)
        ]

        def b_body(b, c):
            v = xcp_v[b, :]
            for cv in range(N_CONT_S):
                bc = _lane_broadcast(v, cv)
                o = cv * EMB_S
                cont_v[b, pl.ds(o, LANES)] = bc * fe_v[cv, pl.ds(0, LANES)]
                cont_v[b, pl.ds(o + LANES, LANES)] = (
                    bc * fe_v[cv, pl.ds(LANES, LANES)])
            return c

        lax.fori_loop(0, CHUNK, b_body, 0)

        for cp in cps:
            cp.wait()

        for f in range(N_FIELDS_S):
            pltpu.async_copy(
                rows_v.at[pl.ds(f * CHUNK, CHUNK)],
                out_hbm.at[pl.ds(c0, CHUNK), pl.ds(f * EMB_S, EMB_S)],
                osem)
        pltpu.async_copy(
            cont_v,
            out_hbm.at[pl.ds(c0, CHUNK), pl.ds(CONT_OFF, N_CONT_S * EMB_S)],
            osem)

    run_chunk(0, 0, False)
    run_chunk(1, 1, False)

    def outer(i, carry):
        run_chunk(2 * i, 0, True)
        run_chunk(2 * i + 1, 1, True)
        return carry

    lax.fori_loop(1, n_chunks // 2, outer, 0)

    drain_out(0)
    drain_out(1)


def kernel(x_sparse, x_cont, tables, field_embeddings):
    B, F = x_sparse.shape
    V, E = tables.shape[1], tables.shape[2]
    C = x_cont.shape[1]
    n_chunks_total = B // CHUNK

    tab3 = _tc_relayout(tables.transpose(0, 2, 1))
    tabf = tab3.reshape(tab3.shape[0] * V * 4, E)
    xs_fm = (x_sparse.T.reshape(F, n_chunks_total, CHUNK)
             .transpose(1, 0, 2).reshape(n_chunks_total, F // 2, 64))
    fld = jnp.arange(F, dtype=jnp.int32)
    bases = (fld // 4) * (4 * V) + (fld % 4)
    offs = jnp.repeat(bases, CHUNK).reshape(F // 2, 64)
    xcp = jnp.pad(x_cont, ((0, 0), (0, LANES - C)))

    mesh = plsc.VectorSubcoreMesh(core_axis_name="c", subcore_axis_name="s")
    run = pl.kernel(
        _body,
        out_type=jax.ShapeDtypeStruct((B, (F + C) * E), jnp.float32),
        mesh=mesh,
        scratch_types=[
            pltpu.VMEM((F // 2, 64), jnp.int32),
            pltpu.VMEM((F // 2, 64), jnp.int32),
            pltpu.VMEM((F * CHUNK, E), jnp.float32),
            pltpu.VMEM((F * CHUNK, E), jnp.float32),
            pltpu.VMEM((CHUNK, C * E), jnp.float32),
            pltpu.VMEM((CHUNK, C * E), jnp.float32),
            pltpu.VMEM((CHUNK, LANES), jnp.float32),
            pltpu.VMEM((CHUNK, LANES), jnp.float32),
            pltpu.VMEM((F // 2, 64), jnp.int32),
            pltpu.VMEM((C, E), jnp.float32),
            pltpu.SemaphoreType.DMA,
            pltpu.SemaphoreType.DMA,
            pltpu.SemaphoreType.DMA,
            pltpu.SemaphoreType.DMA,
        ],
        compiler_params=pltpu.CompilerParams(use_tc_tiling_on_sc=False),
    )
    return run(tabf, xs_fm, offs, xcp, field_embeddings)


# speedup vs baseline: 8.0052x; 1.0301x over previous
import jax
import jax.numpy as jnp
from jax import lax
from jax.experimental import pallas as pl
from jax.experimental.pallas import tpu as pltpu
from jax.experimental.pallas import tpu_sc as plsc

N_FIELDS_S = 26
VOCAB_S = 100000
EMB_S = 32
N_CONT_S = 13
CHUNK = 32
LANES = 16
N_WORKERS = 32
CONT_OFF = N_FIELDS_S * EMB_S
N_GATHERS = N_FIELDS_S * CHUNK // 64
PAIR_ROWS = N_FIELDS_S // 2


def _tc_relayout(tables_t):
    F, E, V = tables_t.shape
    GF = 4
    G = -(-F // GF)
    W = 12544
    grid = (G, pl.cdiv(V, W))

    def body(in_ref, out_ref):
        x = jnp.concatenate([in_ref[a] for a in range(GF)], axis=0)
        out_ref[0] = x.T

    return pl.pallas_call(
        body,
        grid=grid,
        in_specs=[pl.BlockSpec((GF, E, W), lambda g, j: (g, 0, j))],
        out_specs=pl.BlockSpec((1, W, GF * E), lambda g, j: (g, j, 0)),
        out_shape=jax.ShapeDtypeStruct((G, V, GF * E), jnp.float32),
    )(tables_t)


def _lane_broadcast(v, lane):
    idx = jnp.full((LANES, 1), lane, jnp.int32)
    dnums = lax.GatherDimensionNumbers(
        offset_dims=(), collapsed_slice_dims=(0,), start_index_map=(0,))
    return lax.gather(v, idx, dnums, slice_sizes=(1,),
                      mode=lax.GatherScatterMode.PROMISE_IN_BOUNDS)


def _body(tabf_hbm, xs_fm_hbm, offs_hbm, xcp_hbm, fe_hbm, out_hbm,
          idx0, idx1, rows0, rows1, cont0, cont1, xcp0, xcp1, offs_v, fe_v,
          gsem0, gsem1, osem0, osem1):
    B = out_hbm.shape[0]
    rows_per_w = B // N_WORKERS
    n_chunks = rows_per_w // CHUNK

    wid = lax.axis_index("s") * 2 + lax.axis_index("c")
    chunk_base = wid * n_chunks

    pltpu.sync_copy(fe_hbm, fe_v)
    pltpu.sync_copy(offs_hbm, offs_v)

    idx_b = (idx0, idx1)
    rows_b = (rows0, rows1)
    cont_b = (cont0, cont1)
    xcp_b = (xcp0, xcp1)
    gsem_b = (gsem0, gsem1)
    osem_b = (osem0, osem1)

    def drain_out(p):
        rows_v, cont_v, osem = rows_b[p], cont_b[p], osem_b[p]
        for f in range(N_FIELDS_S):
            pltpu.make_async_copy(
                rows_v.at[pl.ds(f * CHUNK, CHUNK)],
                out_hbm.at[pl.ds(0, CHUNK), pl.ds(f * EMB_S, EMB_S)],
                osem).wait()
        pltpu.make_async_copy(
            cont_v,
            out_hbm.at[pl.ds(0, CHUNK), pl.ds(CONT_OFF, N_CONT_S * EMB_S)],
            osem).wait()

    def run_chunk(g, p, wait_out):
        idx_v, rows_v, cont_v = idx_b[p], rows_b[p], cont_b[p]
        xcp_v, gsem, osem = xcp_b[p], gsem_b[p], osem_b[p]
        ck = chunk_base + g
        c0 = ck * CHUNK

        if wait_out:
            drain_out(p)

        pltpu.sync_copy(xs_fm_hbm.at[ck], idx_v)
        pltpu.sync_copy(xcp_hbm.at[pl.ds(c0, CHUNK)], xcp_v)

        for j in range(PAIR_ROWS):
            for k in range(4):
                sl = pl.ds(k * LANES, LANES)
                idx_v[j, sl] = idx_v[j, sl] * 4 + offs_v[j, sl]

        cps = [
            pltpu.async_copy(
                tabf_hbm.at[idx_v.at[j]],
                rows_v.at[pl.ds(j * 64, 64)],
                gsem)
            for j in range(N_GATHERS)
        ]

        def b_body(b, c):
            v = xcp_v[b, :]
            for cv in range(N_CONT_S):
                bc = _lane_broadcast(v, cv)
                o = cv * EMB_S
                cont_v[b, pl.ds(o, LANES)] = bc * fe_v[cv, pl.ds(0, LANES)]
                cont_v[b, pl.ds(o + LANES, LANES)] = (
                    bc * fe_v[cv, pl.ds(LANES, LANES)])
            return c

        lax.fori_loop(0, CHUNK, b_body, 0)

        for cp in cps:
            cp.wait()

        for f in range(N_FIELDS_S):
            pltpu.async_copy(
                rows_v.at[pl.ds(f * CHUNK, CHUNK)],
                out_hbm.at[pl.ds(c0, CHUNK), pl.ds(f * EMB_S, EMB_S)],
                osem)
        pltpu.async_copy(
            cont_v,
            out_hbm.at[pl.ds(c0, CHUNK), pl.ds(CONT_OFF, N_CONT_S * EMB_S)],
            osem)

    run_chunk(0, 0, False)
    run_chunk(1, 1, False)

    def outer(i, carry):
        run_chunk(2 * i, 0, True)
        run_chunk(2 * i + 1, 1, True)
        return carry

    lax.fori_loop(1, n_chunks // 2, outer, 0)

    drain_out(0)
    drain_out(1)


def kernel(x_sparse, x_cont, tables, field_embeddings):
    B, F = x_sparse.shape
    V, E = tables.shape[1], tables.shape[2]
    C = x_cont.shape[1]
    n_chunks_total = B // CHUNK

    tab3 = _tc_relayout(tables.transpose(0, 2, 1))
    tabf = tab3.reshape(tab3.shape[0] * V * 4, E)
    xs_fm = (x_sparse.T.reshape(F, n_chunks_total, CHUNK)
             .transpose(1, 0, 2).reshape(n_chunks_total, F // 2, 64))
    fld = jnp.arange(F, dtype=jnp.int32)
    bases = (fld // 4) * (4 * V) + (fld % 4)
    offs = jnp.repeat(bases, CHUNK).reshape(F // 2, 64)
    xcp = jnp.pad(x_cont, ((0, 0), (0, LANES - C)))

    mesh = plsc.VectorSubcoreMesh(core_axis_name="c", subcore_axis_name="s")
    run = pl.kernel(
        _body,
        out_type=jax.ShapeDtypeStruct((B, (F + C) * E), jnp.float32),
        mesh=mesh,
        scratch_types=[
            pltpu.VMEM((F // 2, 64), jnp.int32),
            pltpu.VMEM((F // 2, 64), jnp.int32),
            pltpu.VMEM((F * CHUNK, E), jnp.float32),
            pltpu.VMEM((F * CHUNK, E), jnp.float32),
            pltpu.VMEM((CHUNK, C * E), jnp.float32),
            pltpu.VMEM((CHUNK, C * E), jnp.float32),
            pltpu.VMEM((CHUNK, LANES), jnp.float32),
            pltpu.VMEM((CHUNK, LANES), jnp.float32),
            pltpu.VMEM((F // 2, 64), jnp.int32),
            pltpu.VMEM((C, E), jnp.float32),
            pltpu.SemaphoreType.DMA,
            pltpu.SemaphoreType.DMA,
            pltpu.SemaphoreType.DMA,
            pltpu.SemaphoreType.DMA,
        ],
        compiler_params=pltpu.CompilerParams(use_tc_tiling_on_sc=False),
    )
    return run(tabf, xs_fm, offs, xcp, field_embeddings)
